# baseline (speedup 1.0000x reference)
import functools

import jax
import jax.numpy as jnp
from jax import lax
from jax.experimental import pallas as pl
from jax.experimental.pallas import tpu as pltpu
from jax.experimental.pallas import tpu_sc as plsc

VOCAB = 1000
EMBED_DIM = 64
BATCH = 16384

NUM_CORES = 2
NUM_SUBCORES = 16
NUM_WORKERS = NUM_CORES * NUM_SUBCORES
B_PER_W = BATCH // NUM_WORKERS
LANES = 16
N_GROUPS = B_PER_W // LANES
N_BUFS = 4
GROUPS_PER_BUF = N_GROUPS // N_BUFS


@functools.cache
def _emb_kernel():
    mesh = plsc.VectorSubcoreMesh(
        core_axis_name="c", subcore_axis_name="s",
        num_cores=NUM_CORES, num_subcores=NUM_SUBCORES)

    @functools.partial(
        pl.kernel,
        mesh=mesh,
        out_type=jax.ShapeDtypeStruct((EMBED_DIM, BATCH), jnp.float32),
        scratch_types=[
            pltpu.VMEM((EMBED_DIM, VOCAB), jnp.float32),
            pltpu.VMEM((EMBED_DIM,), jnp.float32),
            pltpu.VMEM((B_PER_W,), jnp.int32),
        ] + [pltpu.VMEM((EMBED_DIM, GROUPS_PER_BUF * LANES), jnp.float32)
             for _ in range(N_BUFS)],
        compiler_params=pltpu.CompilerParams(
            use_tc_tiling_on_sc=True, needs_layout_passes=False),
    )
    def body(w_hbm, b_hbm, idx_hbm, out_hbm, w_v, b_v, idx_v, *bufs):
        wid = lax.axis_index("s") * NUM_CORES + lax.axis_index("c")
        base = wid * B_PER_W
        pltpu.sync_copy(idx_hbm.at[pl.ds(base, B_PER_W)], idx_v)
        pltpu.sync_copy(w_hbm, w_v)
        pltpu.sync_copy(b_hbm, b_v)

        def dbody(d, carry):
            d16 = jnp.full((LANES,), d, jnp.int32)
            bias = plsc.load_gather(b_v, [d16])
            for k in range(N_GROUPS):
                i16 = idx_v[pl.ds(k * LANES, LANES)]
                v = plsc.load_gather(w_v, [d16, i16])
                buf = bufs[k // GROUPS_PER_BUF]
                buf[d, pl.ds((k % GROUPS_PER_BUF) * LANES, LANES)] = v + bias
            return carry

        lax.fori_loop(0, EMBED_DIM, dbody, 0)
        for kk in range(N_BUFS):
            pltpu.sync_copy(
                bufs[kk],
                out_hbm.at[:, pl.ds(base + kk * GROUPS_PER_BUF * LANES,
                                    GROUPS_PER_BUF * LANES)])

    return body


def kernel(x, W, b):
    idx = x.astype(jnp.int32)
    out_t = _emb_kernel()(W, b, idx)
    return out_t.T


# speedup vs baseline: 1.4417x; 1.4417x over previous
import functools

import jax
import jax.numpy as jnp
from jax import lax
from jax.experimental import pallas as pl
from jax.experimental.pallas import tpu as pltpu
from jax.experimental.pallas import tpu_sc as plsc

VOCAB = 1000
EMBED_DIM = 64
BATCH = 16384

NUM_CORES = 2
NUM_SUBCORES = 16
NUM_WORKERS = NUM_CORES * NUM_SUBCORES
LANES = 16

D_GROUPS = 8
D_PER_G = EMBED_DIM // D_GROUPS
Q_SPLITS = NUM_WORKERS // D_GROUPS
B_PER_Q = BATCH // Q_SPLITS
N_CHUNKS = B_PER_Q // 128
GROUPS_PER_CHUNK = 128 // LANES


@functools.cache
def _emb_kernel():
    mesh = plsc.VectorSubcoreMesh(
        core_axis_name="c", subcore_axis_name="s",
        num_cores=NUM_CORES, num_subcores=NUM_SUBCORES)

    @functools.partial(
        pl.kernel,
        mesh=mesh,
        out_type=jax.ShapeDtypeStruct((EMBED_DIM, BATCH), jnp.float32),
        scratch_types=[
            pltpu.VMEM((D_PER_G * 1024,), jnp.float32),
            pltpu.VMEM((EMBED_DIM,), jnp.float32),
            pltpu.VMEM((B_PER_Q,), jnp.int32),
            pltpu.VMEM((D_PER_G, B_PER_Q), jnp.float32),
            pltpu.SemaphoreType.DMA,
            pltpu.SemaphoreType.DMA,
        ],
        compiler_params=pltpu.CompilerParams(
            use_tc_tiling_on_sc=True, needs_layout_passes=False),
    )
    def body(w_hbm, b_hbm, idx_hbm, out_hbm, w_v, b_v, idx_v, stage_v,
             sem_in, sem):
        wid = lax.axis_index("s") * NUM_CORES + lax.axis_index("c")
        g = wid // Q_SPLITS
        q = wid % Q_SPLITS
        pre = [pltpu.async_copy(idx_hbm.at[pl.ds(q * B_PER_Q, B_PER_Q)],
                                idx_v, sem_in),
               pltpu.async_copy(b_hbm, b_v, sem_in)]
        for dl in range(D_PER_G):
            for l in range(8):
                pre.append(pltpu.async_copy(
                    w_hbm.at[g * D_PER_G + dl, pl.ds(l * 128, 128)],
                    w_v.at[pl.ds(dl * 1024 + l * 128, 128)],
                    sem_in))
        for c in pre:
            c.wait()
        biases = [
            plsc.load_gather(b_v, [jnp.full((LANES,), g * D_PER_G + dl,
                                            jnp.int32)])
            for dl in range(D_PER_G)
        ]

        copies = []
        for kk in range(N_CHUNKS):
            def kbody(k8, carry, kk=kk):
                col = kk * 128 + k8 * LANES
                i16 = idx_v[pl.ds(col, LANES)]
                for dl in range(D_PER_G):
                    f = i16 + jnp.int32(dl * 1024)
                    v = plsc.load_gather(w_v, [f])
                    stage_v[dl, pl.ds(col, LANES)] = v + biases[dl]
                return carry

            lax.fori_loop(0, GROUPS_PER_CHUNK, kbody, 0)
            copies.append(pltpu.async_copy(
                stage_v.at[:, pl.ds(kk * 128, 128)],
                out_hbm.at[pl.ds(g * D_PER_G, D_PER_G),
                           pl.ds(q * B_PER_Q + kk * 128, 128)],
                sem,
            ))
        for c in copies:
            c.wait()

    return body


def kernel(x, W, b):
    idx = x.astype(jnp.int32)
    w_pad = jnp.pad(W, ((0, 0), (0, 1024 - VOCAB)))
    out_t = _emb_kernel()(w_pad, b, idx)
    return out_t.T


# speedup vs baseline: 1.6694x; 1.1579x over previous
import functools

import jax
import jax.numpy as jnp
from jax import lax
from jax.experimental import pallas as pl
from jax.experimental.pallas import tpu as pltpu
from jax.experimental.pallas import tpu_sc as plsc

VOCAB = 1000
V_PAD = 1024
EMBED_DIM = 64
BATCH = 16384

NUM_CORES = 2
NUM_SUBCORES = 16
NUM_WORKERS = NUM_CORES * NUM_SUBCORES
LANES = 16

D_GROUPS = 8
D_PER_G = EMBED_DIM // D_GROUPS
Q_SPLITS = NUM_WORKERS // D_GROUPS
B_PER_Q = BATCH // Q_SPLITS
N_K = B_PER_Q // LANES
L_TILES = V_PAD // 128


@functools.cache
def _emb_kernel():
    mesh = plsc.VectorSubcoreMesh(
        core_axis_name="c", subcore_axis_name="s",
        num_cores=NUM_CORES, num_subcores=NUM_SUBCORES)

    @functools.partial(
        pl.kernel,
        mesh=mesh,
        out_type=jax.ShapeDtypeStruct((EMBED_DIM, BATCH), jnp.float32),
        scratch_types=[
            pltpu.VMEM((L_TILES, D_PER_G, 128), jnp.float32),
            pltpu.VMEM((EMBED_DIM,), jnp.float32),
            pltpu.VMEM((B_PER_Q,), jnp.int32),
            pltpu.VMEM((D_PER_G, B_PER_Q), jnp.float32),
            pltpu.SemaphoreType.DMA,
        ],
        compiler_params=pltpu.CompilerParams(
            use_tc_tiling_on_sc=True, needs_layout_passes=False),
    )
    def body(w_hbm, b_hbm, idx_hbm, out_hbm, w_v, b_v, idx_v, stage_v, sem):
        wid = lax.axis_index("s") * NUM_CORES + lax.axis_index("c")
        g = wid // Q_SPLITS
        q = wid % Q_SPLITS
        pre = [pltpu.async_copy(idx_hbm.at[pl.ds(q * B_PER_Q, B_PER_Q)],
                                idx_v, sem),
               pltpu.async_copy(b_hbm, b_v, sem)]
        for l in range(L_TILES):
            pre.append(pltpu.async_copy(
                w_hbm.at[pl.ds(g * D_PER_G, D_PER_G), pl.ds(l * 128, 128)],
                w_v.at[l], sem))
        for c in pre:
            c.wait()
        biases = [
            plsc.load_gather(b_v, [jnp.full((LANES,), g * D_PER_G + dl,
                                            jnp.int32)])
            for dl in range(D_PER_G)
        ]
        dls = [jnp.full((LANES,), dl, jnp.int32) for dl in range(D_PER_G)]

        def kbody(k, carry):
            col = k * LANES
            i16 = idx_v[pl.ds(col, LANES)]
            l16 = lax.shift_right_logical(i16, 7)
            m16 = lax.bitwise_and(i16, jnp.int32(127))
            for dl in range(D_PER_G):
                v = plsc.load_gather(w_v, [l16, dls[dl], m16])
                stage_v[dl, pl.ds(col, LANES)] = v + biases[dl]
            return carry

        lax.fori_loop(0, N_K, kbody, 0)
        pltpu.sync_copy(stage_v,
                        out_hbm.at[pl.ds(g * D_PER_G, D_PER_G),
                                   pl.ds(q * B_PER_Q, B_PER_Q)])

    return body


def kernel(x, W, b):
    idx = x.astype(jnp.int32)
    w_pad = jnp.pad(W, ((0, 0), (0, V_PAD - VOCAB)))
    out_t = _emb_kernel()(w_pad, b, idx)
    return out_t.T


# speedup vs baseline: 2.3906x; 1.4320x over previous
import functools

import jax
import jax.numpy as jnp
from jax import lax
from jax.experimental import pallas as pl
from jax.experimental.pallas import tpu as pltpu
from jax.experimental.pallas import tpu_sc as plsc

VOCAB = 1000
V_PAD = 1024
EMBED_DIM = 64
BATCH = 16384

NUM_CORES = 2
NUM_SUBCORES = 16
NUM_WORKERS = NUM_CORES * NUM_SUBCORES
LANES = 16

D_GROUPS = 8
D_PER_G = EMBED_DIM // D_GROUPS
Q_SPLITS = NUM_WORKERS // D_GROUPS
B_PER_Q = BATCH // Q_SPLITS
N_K = B_PER_Q // LANES
L_TILES = V_PAD // 128


@functools.cache
def _emb_kernel():
    mesh = plsc.VectorSubcoreMesh(
        core_axis_name="c", subcore_axis_name="s",
        num_cores=NUM_CORES, num_subcores=NUM_SUBCORES)

    @functools.partial(
        pl.kernel,
        mesh=mesh,
        out_type=jax.ShapeDtypeStruct((EMBED_DIM, BATCH), jnp.float32),
        scratch_types=[
            pltpu.VMEM((L_TILES, D_PER_G, 128), jnp.float32),
            pltpu.VMEM((EMBED_DIM,), jnp.float32),
            pltpu.VMEM((B_PER_Q,), jnp.int32),
            pltpu.VMEM((D_PER_G, B_PER_Q), jnp.float32),
            pltpu.SemaphoreType.DMA,
        ],
        compiler_params=pltpu.CompilerParams(
            use_tc_tiling_on_sc=True, needs_layout_passes=False),
    )
    def body(w_hbm, b_hbm, idx_hbm, out_hbm, w_v, b_v, idx_v, stage_v, sem):
        wid = lax.axis_index("s") * NUM_CORES + lax.axis_index("c")
        g = wid // Q_SPLITS
        q = wid % Q_SPLITS
        pre = [pltpu.async_copy(idx_hbm.at[pl.ds(q * B_PER_Q, B_PER_Q)],
                                idx_v, sem),
               pltpu.async_copy(b_hbm, b_v, sem)]
        for l in range(L_TILES):
            pre.append(pltpu.async_copy(
                w_hbm.at[pl.ds(g * D_PER_G, D_PER_G), pl.ds(l * 128, 128)],
                w_v.at[l], sem))
        for c in pre:
            c.wait()
        biases = [
            plsc.load_gather(b_v, [jnp.full((LANES,), g * D_PER_G + dl,
                                            jnp.int32)])
            for dl in range(D_PER_G)
        ]
        dls = [jnp.full((LANES,), dl, jnp.int32) for dl in range(D_PER_G)]

        @plsc.parallel_loop(0, N_K, step=1, unroll=4)
        def kbody(k):
            col = k * LANES
            i16 = idx_v[pl.ds(col, LANES)]
            l16 = lax.shift_right_logical(i16, 7)
            m16 = lax.bitwise_and(i16, jnp.int32(127))
            for dl in range(D_PER_G):
                v = plsc.load_gather(w_v, [l16, dls[dl], m16])
                stage_v[dl, pl.ds(col, LANES)] = v + biases[dl]
        pltpu.sync_copy(stage_v,
                        out_hbm.at[pl.ds(g * D_PER_G, D_PER_G),
                                   pl.ds(q * B_PER_Q, B_PER_Q)])

    return body


def kernel(x, W, b):
    idx = x.astype(jnp.int32)
    w_pad = jnp.pad(W, ((0, 0), (0, V_PAD - VOCAB)))
    out_t = _emb_kernel()(w_pad, b, idx)
    return out_t.T
